# baseline (speedup 1.0000x reference)
import functools

import jax
import jax.numpy as jnp
from jax import lax
from jax.experimental import pallas as pl
from jax.experimental.pallas import tpu as pltpu
from jax.experimental.pallas import tpu_sc as plsc


def _make_sc_kernel(B, L, E, NC, NS):
    NW = NC * NS
    BLK = 128
    assert B == NW * BLK
    EO, ES = E // 8, 8
    assert EO * ES == E and L % 2 == 0

    mesh = plsc.VectorSubcoreMesh(core_axis_name="c", subcore_axis_name="s")

    @functools.partial(
        pl.kernel,
        out_type=jax.ShapeDtypeStruct((L, EO, NW, ES, BLK), jnp.float32),
        mesh=mesh,
        scratch_types=[
            pltpu.VMEM((L, E), jnp.float32),
            pltpu.VMEM((BLK, L), jnp.int32),
            pltpu.VMEM((L, BLK), jnp.int32),
            pltpu.VMEM((2, BLK, E), jnp.float32),
            pltpu.VMEM((2, EO, ES, BLK), jnp.float32),
            pltpu.SemaphoreType.DMA((2,)),
            pltpu.SemaphoreType.DMA((2,)),
        ],
        compiler_params=pltpu.CompilerParams(
            use_tc_tiling_on_sc=False, needs_layout_passes=False),
    )
    def emb(tok_hbm, idx_hbm, pos_hbm, out_hbm, pos_v, idxr_v, idxt_v,
            rows_v, outb_v, gsem, osem):
        wid = lax.axis_index("s") * NC + lax.axis_index("c")
        b0 = wid * BLK
        pltpu.sync_copy(idx_hbm.at[pl.ds(b0, BLK), :], idxr_v)
        pltpu.sync_copy(pos_hbm, pos_v)
        lanes = lax.iota(jnp.int32, 16)

        @pl.loop(0, L)
        def _transpose_idx(l):
            lvec = lanes * 0 + l
            for j in range(BLK // 16):
                idxt_v[l, pl.ds(j * 16, 16)] = plsc.load_gather(
                    idxr_v, [lanes + (j * 16), lvec])

        def fire_gather(l, s):
            return pltpu.async_copy(
                tok_hbm.at[idxt_v.at[l, :]], rows_v.at[s], gsem.at[s])

        fire_gather(0, 0)
        fire_gather(1, 1)

        @pl.loop(0, L // 2)
        def _unit(g):
            for s in (0, 1):
                l = g * 2 + s
                pltpu.make_async_copy(
                    tok_hbm.at[pl.ds(0, BLK), :], rows_v.at[s], gsem.at[s]
                ).wait()

                @pl.when(g > 0)
                def _():
                    pltpu.make_async_copy(
                        outb_v.at[s], out_hbm.at[0, :, 0, :, :], osem.at[s]
                    ).wait()

                phalf = [pos_v[l, pl.ds(0, 16)], pos_v[l, pl.ds(16, 16)]]
                for eo in range(EO):
                    for es in range(ES):
                        e = eo * ES + es
                        p = phalf[e // 16][e % 16]
                        evec = lanes * 0 + e
                        for j in range(BLK // 16):
                            v = plsc.load_gather(
                                rows_v.at[s], [lanes + (j * 16), evec])
                            outb_v[s, eo, es, pl.ds(j * 16, 16)] = v + p

                @pl.when(g < L // 2 - 1)
                def _():
                    fire_gather(l + 2, s)

                pltpu.async_copy(
                    outb_v.at[s], out_hbm.at[l, :, wid, :, :], osem.at[s])

        for s in (0, 1):
            pltpu.make_async_copy(
                outb_v.at[s], out_hbm.at[0, :, 0, :, :], osem.at[s]).wait()

    return emb


def kernel(inputs, tok_table, pos_table):
    B, L = inputs.shape
    E = tok_table.shape[1]
    info = plsc.get_sparse_core_info()
    emb = _make_sc_kernel(B, L, E, info.num_cores, info.num_subcores)
    out5 = emb(tok_table, inputs.astype(jnp.int32), pos_table)
    return out5.transpose(2, 4, 0, 1, 3).reshape(B, L, E)
